# baseline (speedup 1.0000x reference)
import functools

import jax
import jax.numpy as jnp
from jax import lax
from jax.experimental import pallas as pl
from jax.experimental.pallas import tpu as pltpu
from jax.experimental.pallas import tpu_sc as plsc

B = 16384
D = 64
NC, NS = 2, 16
NW = NC * NS
BPW = B // NW


def _norm_age_body(x_ref, o_ref):
    x = x_ref[...]
    n = x.size
    mean = jnp.sum(x) / n
    d = x - mean
    var = jnp.sum(d * d) / (n - 1)
    o_ref[...] = d / (jnp.sqrt(var) + 1e-6)


_norm_age = pl.pallas_call(
    _norm_age_body,
    out_shape=jax.ShapeDtypeStruct((128, 128), jnp.float32),
)


def _sc_body(w_sex, w_tl, w_msi, w_st, sex, tl, msi, st, nage, out,
             idx_v, rows_v, age_v, sem):
    wid = lax.axis_index("s") * NC + lax.axis_index("c")
    base = wid * BPW
    for w_ref, i_ref, off in ((w_sex, sex, 0), (w_tl, tl, 64),
                              (w_msi, msi, 128), (w_st, st, 192)):
        pltpu.sync_copy(i_ref.at[pl.ds(base, BPW)], idx_v)
        pltpu.async_copy(w_ref.at[idx_v], rows_v, sem).wait()
        pltpu.sync_copy(rows_v, out.at[pl.ds(base, BPW), pl.ds(off, D)])
    pltpu.sync_copy(nage.at[pl.ds(base, BPW)], age_v)
    pltpu.sync_copy(age_v, out.at[pl.ds(base, BPW), pl.ds(4 * D, 1)])


_sc_lookup = functools.partial(
    pl.kernel,
    mesh=plsc.VectorSubcoreMesh(core_axis_name="c", subcore_axis_name="s"),
    out_type=jax.ShapeDtypeStruct((B, 4 * D + 1), jnp.float32),
    scratch_types=[
        pltpu.VMEM((BPW,), jnp.int32),
        pltpu.VMEM((BPW, D), jnp.float32),
        pltpu.VMEM((BPW, 1), jnp.float32),
        pltpu.SemaphoreType.DMA,
    ],
    compiler_params=pltpu.CompilerParams(use_tc_tiling_on_sc=False),
)(_sc_body)


def kernel(sex, tumor_location, msi_status, stage, age,
           W_sex, W_tumor_location, W_msi_status, W_stage):
    nage = _norm_age(age.reshape(128, 128)).reshape(B, 1)
    return _sc_lookup(W_sex, W_tumor_location, W_msi_status, W_stage,
                      sex, tumor_location, msi_status, stage, nage)


# speedup vs baseline: 1.4107x; 1.4107x over previous
import functools

import jax
import jax.numpy as jnp
from jax import lax
from jax.experimental import pallas as pl
from jax.experimental.pallas import tpu as pltpu
from jax.experimental.pallas import tpu_sc as plsc

B = 16384
D = 64
NC, NS = 2, 16
NW = NC * NS
BPW = B // NW


def _norm_age_body(x_ref, o_ref):
    x = x_ref[...]
    n = x.size
    mean = jnp.sum(x) / n
    d = x - mean
    var = jnp.sum(d * d) / (n - 1)
    o_ref[...] = d / (jnp.sqrt(var) + 1e-6)


_norm_age = pl.pallas_call(
    _norm_age_body,
    out_shape=jax.ShapeDtypeStruct((128, 128), jnp.float32),
)


def _sc_body(w_sex, w_tl, w_msi, w_st, sex, tl, msi, st, nage, out,
             i0, i1, i2, i3, b0, b1, b2, age_v,
             s0, s1, s2, s3, g0, g1, g2, ws0, ws1, ws2, asem):
    wid = lax.axis_index("s") * NC + lax.axis_index("c")
    base = wid * BPW
    idx_v = (i0, i1, i2, i3)
    bufs = (b0, b1, b2)
    gsems = (g0, g1, g2)
    wsems = (ws0, ws1, ws2)
    tables = (w_sex, w_tl, w_msi, w_st)

    icopies = [
        pltpu.async_copy(src.at[pl.ds(base, BPW)], dst, sem)
        for src, dst, sem in zip((sex, tl, msi, st), idx_v, (s0, s1, s2, s3))
    ]
    acopy = pltpu.async_copy(nage.at[pl.ds(base, BPW)], age_v, asem)

    def gather(f):
        icopies[f].wait()
        return pltpu.async_copy(tables[f].at[idx_v[f]], bufs[f % 3],
                                gsems[f % 3])

    def write(f):
        return pltpu.async_copy(
            bufs[f % 3], out.at[pl.ds(base, BPW), pl.ds(D * f, D)],
            wsems[f % 3])

    ga = [gather(0), gather(1), gather(2)]
    ga[0].wait()
    w0 = write(0)
    w0.wait()
    ga.append(gather(3))
    ga[1].wait()
    w1 = write(1)
    ga[2].wait()
    w2 = write(2)
    ga[3].wait()
    w3 = write(3)
    acopy.wait()
    wa = pltpu.async_copy(age_v, out.at[pl.ds(base, BPW), pl.ds(4 * D, 1)],
                          asem)
    w1.wait()
    w2.wait()
    w3.wait()
    wa.wait()


_sc_lookup = functools.partial(
    pl.kernel,
    mesh=plsc.VectorSubcoreMesh(core_axis_name="c", subcore_axis_name="s"),
    out_type=jax.ShapeDtypeStruct((B, 4 * D + 1), jnp.float32),
    scratch_types=[
        pltpu.VMEM((BPW,), jnp.int32),
        pltpu.VMEM((BPW,), jnp.int32),
        pltpu.VMEM((BPW,), jnp.int32),
        pltpu.VMEM((BPW,), jnp.int32),
        pltpu.VMEM((BPW, D), jnp.float32),
        pltpu.VMEM((BPW, D), jnp.float32),
        pltpu.VMEM((BPW, D), jnp.float32),
        pltpu.VMEM((BPW, 1), jnp.float32),
        pltpu.SemaphoreType.DMA,
        pltpu.SemaphoreType.DMA,
        pltpu.SemaphoreType.DMA,
        pltpu.SemaphoreType.DMA,
        pltpu.SemaphoreType.DMA,
        pltpu.SemaphoreType.DMA,
        pltpu.SemaphoreType.DMA,
        pltpu.SemaphoreType.DMA,
        pltpu.SemaphoreType.DMA,
        pltpu.SemaphoreType.DMA,
        pltpu.SemaphoreType.DMA,
    ],
    compiler_params=pltpu.CompilerParams(use_tc_tiling_on_sc=False),
)(_sc_body)


def kernel(sex, tumor_location, msi_status, stage, age,
           W_sex, W_tumor_location, W_msi_status, W_stage):
    nage = _norm_age(age.reshape(128, 128)).reshape(B, 1)
    return _sc_lookup(W_sex, W_tumor_location, W_msi_status, W_stage,
                      sex, tumor_location, msi_status, stage, nage)


# speedup vs baseline: 3.9790x; 2.8207x over previous
import functools

import jax
import jax.numpy as jnp
from jax import lax
from jax.experimental import pallas as pl
from jax.experimental.pallas import tpu as pltpu
from jax.experimental.pallas import tpu_sc as plsc

B = 16384
D = 64
NC, NS = 2, 16
NW = NC * NS
BPW = B // NW
ROWS_PER_STEP = 2048


def _sc_body(w_sex, w_tl, w_msi, w_st, sex, tl, msi, st, nage, out,
             i0, i1, i2, i3, b0, b1, b2, t0, t1, t2, t3, age_h, age_v,
             s0, s1, s2, s3, g0, g1, g2, ws0, ws1, ws2, tsem, asem):
    wid = lax.axis_index("s") * NC + lax.axis_index("c")
    base = wid * BPW
    idx_v = (i0, i1, i2, i3)
    bufs = (b0, b1, b2)
    gsems = (g0, g1, g2)
    wsems = (ws0, ws1, ws2)
    spm = (t0, t1, t2, t3)

    @pl.when(lax.axis_index("s") == 0)
    def _():
        for w_ref, t_ref in zip((w_sex, w_tl, w_msi, w_st), spm):
            pltpu.async_copy(w_ref, t_ref, tsem).wait()

    icopies = [
        pltpu.async_copy(src.at[pl.ds(base, BPW)], dst, sem)
        for src, dst, sem in zip((sex, tl, msi, st), idx_v, (s0, s1, s2, s3))
    ]
    acopy = pltpu.async_copy(nage.at[pl.ds(wid, 1), :], age_h, asem)
    plsc.subcore_barrier()

    def gather(f):
        icopies[f].wait()
        return pltpu.async_copy(spm[f].at[idx_v[f]], bufs[f % 3],
                                gsems[f % 3])

    def write(f):
        return pltpu.async_copy(
            bufs[f % 3], out.at[pl.ds(base, BPW), pl.ds(D * f, D)],
            wsems[f % 3])

    ga = [gather(0), gather(1), gather(2)]
    ga[0].wait()
    w0 = write(0)
    w0.wait()
    ga.append(gather(3))
    ga[1].wait()
    w1 = write(1)
    ga[2].wait()
    w2 = write(2)
    ga[3].wait()
    w3 = write(3)
    acopy.wait()
    lane = lax.iota(jnp.int32, 16)
    zero = lane * 0
    for k in range(BPW // 16):
        v = age_h[0, pl.ds(16 * k, 16)]
        plsc.store_scatter(age_v, [lane + 16 * k, zero], v)
    wa = pltpu.async_copy(age_v, out.at[pl.ds(base, BPW), pl.ds(4 * D, 1)],
                          asem)
    w1.wait()
    w2.wait()
    w3.wait()
    wa.wait()


_sc_lookup = functools.partial(
    pl.kernel,
    mesh=plsc.VectorSubcoreMesh(core_axis_name="c", subcore_axis_name="s"),
    out_type=jax.ShapeDtypeStruct((B, 4 * D + 1), jnp.float32),
    scratch_types=[
        pltpu.VMEM((BPW,), jnp.int32),
        pltpu.VMEM((BPW,), jnp.int32),
        pltpu.VMEM((BPW,), jnp.int32),
        pltpu.VMEM((BPW,), jnp.int32),
        pltpu.VMEM((BPW, D), jnp.float32),
        pltpu.VMEM((BPW, D), jnp.float32),
        pltpu.VMEM((BPW, D), jnp.float32),
        pltpu.VMEM_SHARED((10, D), jnp.float32),
        pltpu.VMEM_SHARED((10, D), jnp.float32),
        pltpu.VMEM_SHARED((10, D), jnp.float32),
        pltpu.VMEM_SHARED((10, D), jnp.float32),
        pltpu.VMEM((1, BPW), jnp.float32),
        pltpu.VMEM((BPW, 1), jnp.float32),
        pltpu.SemaphoreType.DMA,
        pltpu.SemaphoreType.DMA,
        pltpu.SemaphoreType.DMA,
        pltpu.SemaphoreType.DMA,
        pltpu.SemaphoreType.DMA,
        pltpu.SemaphoreType.DMA,
        pltpu.SemaphoreType.DMA,
        pltpu.SemaphoreType.DMA,
        pltpu.SemaphoreType.DMA,
        pltpu.SemaphoreType.DMA,
        pltpu.SemaphoreType.DMA,
        pltpu.SemaphoreType.DMA,
    ],
    compiler_params=pltpu.CompilerParams(use_tc_tiling_on_sc=False,
                                         needs_layout_passes=False),
)(_sc_body)


def _norm_age_body(x_ref, o_ref):
    x = x_ref[...]
    n = x.size
    mean = jnp.sum(x) / n
    d = x - mean
    var = jnp.sum(d * d) / (n - 1)
    o_ref[...] = d / (jnp.sqrt(var) + 1e-6)


_norm_age = pl.pallas_call(
    _norm_age_body,
    out_shape=jax.ShapeDtypeStruct((128, 128), jnp.float32),
)


def kernel(sex, tumor_location, msi_status, stage, age,
           W_sex, W_tumor_location, W_msi_status, W_stage):
    nage = _norm_age(age.reshape(128, 128)).reshape(NW, BPW)
    return _sc_lookup(W_sex, W_tumor_location, W_msi_status, W_stage,
                      sex, tumor_location, msi_status, stage, nage)
